# baseline (speedup 1.0000x reference)
import jax
import jax.numpy as jnp
from jax import lax
from jax.experimental import pallas as pl
from jax.experimental.pallas import tpu as pltpu
from jax.experimental.pallas import tpu_sc as plsc

N = 100000
NP = 102400
E = 3200000

_NW = 32
_EPW = E // _NW
_CH = 5000
_ZCH = NP // 16

_BNT = 5120
_BET = 25600

_SCALE = 1.0507009873554805
_ALPHA = 1.6732632423543772


def _selu(x):
    return _SCALE * jnp.where(x > 0, x, _ALPHA * (jnp.exp(x) - 1.0))


def _dot(w, x):
    return jnp.dot(w, x, preferred_element_type=jnp.float32)


def _full(shape):
    idx = tuple(0 for _ in shape)
    return pl.BlockSpec(shape, lambda *_, _idx=idx: _idx)


def _rows(mats):
    return jnp.concatenate([m.reshape(1, -1) for m in mats], axis=0)


_BNR = N // (NP // _BNT)


def _prep_body(x_ref, xr_ref,
               wp0, bp0, wp1, bp1, wp2, bp2, wp3, bp3,
               wr0, br0, wr1, br1, wr2, br2, wr3, br3,
               tbl_ref, dec_ref):
    x = x_ref[...]
    h = _selu(_dot(wp0[...], x) + bp0[...])
    h = _selu(_dot(wp1[...], h) + bp1[...])
    h = _selu(_dot(wp2[...], h) + bp2[...])
    ln = _dot(wp3[...], h) + bp3[...]
    z = jnp.zeros((2, x.shape[1]), jnp.float32)
    tbl_ref[...] = jnp.concatenate([ln, x, z], axis=0)
    xr = xr_ref[...]
    h = _selu(jnp.dot(xr, wr0[...], preferred_element_type=jnp.float32)
              + br0[...])
    h = _selu(jnp.dot(h, wr1[...], preferred_element_type=jnp.float32)
              + br1[...])
    h = _selu(jnp.dot(h, wr2[...], preferred_element_type=jnp.float32)
              + br2[...])
    dec_ref[...] = (jnp.dot(h, wr3[...], preferred_element_type=jnp.float32)
                    + br3[...])


def _tc_prep(nodes_t, nodes_row, flat_w):
    specs = [pl.BlockSpec((3, _BNT), lambda i: (0, i)),
             pl.BlockSpec((_BNR, 3), lambda i: (i, 0))]
    for a in flat_w:
        specs.append(_full(a.shape))
    return pl.pallas_call(
        _prep_body,
        grid=(NP // _BNT,),
        in_specs=specs,
        out_specs=[pl.BlockSpec((8, _BNT), lambda i: (0, i)),
                   pl.BlockSpec((_BNR, 10), lambda i: (i, 0))],
        out_shape=[jax.ShapeDtypeStruct((8, NP), jnp.float32),
                   jax.ShapeDtypeStruct((N, 10), jnp.float32)],
    )(nodes_t, nodes_row, *flat_w)


def _gather_body(*refs):
    cols = refs[0:6]
    recv = refs[6]
    send = refs[7]
    out_r = refs[8:14]
    out_s = refs[14:20]
    shc = refs[20:26]
    idx_r = refs[26]
    idx_s = refs[27]
    v_r = refs[28:34]
    v_s = refs[34:40]
    stage = refs[40]
    sem = refs[41]

    c = lax.axis_index("c")
    s = lax.axis_index("s")
    st = pl.ds(s * _ZCH, _ZCH)
    for j in range(6):
        pltpu.sync_copy(cols[j].at[st], stage)
        pltpu.sync_copy(stage, shc[j].at[st])
    plsc.subcore_barrier()
    base = (s * 2 + c) * _EPW

    def step(t, carry):
        off = base + t * _CH
        pltpu.sync_copy(recv.at[pl.ds(off, _CH)], idx_r)
        pltpu.sync_copy(send.at[pl.ds(off, _CH)], idx_s)
        cps = []
        for j in range(6):
            cps.append(pltpu.async_copy(shc[j].at[idx_r], v_r[j], sem))
            cps.append(pltpu.async_copy(shc[j].at[idx_s], v_s[j], sem))
        for cp in cps:
            cp.wait()
        for j in range(6):
            pltpu.sync_copy(v_r[j], out_r[j].at[pl.ds(off, _CH)])
            pltpu.sync_copy(v_s[j], out_s[j].at[pl.ds(off, _CH)])
        return carry

    lax.fori_loop(0, _EPW // _CH, step, 0)


def _sc_gather(cols, recv, send):
    f = pl.kernel(
        _gather_body,
        out_type=[jax.ShapeDtypeStruct((E,), jnp.float32)] * 12,
        mesh=plsc.VectorSubcoreMesh(core_axis_name="c", subcore_axis_name="s"),
        scratch_types=(
            [pltpu.VMEM_SHARED((NP,), jnp.float32)] * 6
            + [pltpu.VMEM((_CH,), jnp.int32)] * 2
            + [pltpu.VMEM((_CH,), jnp.float32)] * 12
            + [pltpu.VMEM((_ZCH,), jnp.float32),
               pltpu.SemaphoreType.DMA]),
    )
    return f(*cols, recv, send)


def _edge_body(*refs):
    x_ref = refs[0]
    r = refs[1:7]
    s = refs[7:13]
    (we0, be0, we1, be1, we2, be2, we3, be3,
     w1, b1, w2, b2, w3, b3) = refs[13:27]
    outs = refs[27:33]

    x = x_ref[...]
    h = _selu(_dot(we0[...], x) + be0[...])
    h = _selu(_dot(we1[...], h) + be1[...])
    h = _selu(_dot(we2[...], h) + be2[...])
    pe3 = _dot(we3[...], h) + be3[...]
    r6 = _rows([q[...] for q in r])
    s6 = _rows([q[...] for q in s])
    ef = jnp.concatenate([pe3, x, r6, s6], axis=0)
    h1 = _selu(_dot(w1[...], ef) + b1[...])
    h2 = _selu(_dot(w2[...], h1) + b2[...])
    ut = _dot(w3[...], h2) + b3[...]
    for j in range(6):
        outs[j][...] = ut[j, :]


def _tc_edge(edges_t, g12, flat_w):
    specs = [pl.BlockSpec((3, _BET), lambda i: (0, i))]
    specs += [pl.BlockSpec((_BET,), lambda i: (i,))] * 12
    for a in flat_w:
        specs.append(_full(a.shape))
    return pl.pallas_call(
        _edge_body,
        grid=(E // _BET,),
        in_specs=specs,
        out_specs=[pl.BlockSpec((_BET,), lambda i: (i,))] * 6,
        out_shape=[jax.ShapeDtypeStruct((E,), jnp.float32)] * 6,
    )(edges_t, *g12, *flat_w)


def _scatter_body(*refs):
    ucols = refs[0:6]
    recv = refs[6]
    zeros = refs[7]
    out_a = refs[8:14]
    out_b = refs[14:20]
    sha = refs[20:26]
    idx_v = refs[26]
    val_v = refs[27]
    stage = refs[28]

    c = lax.axis_index("c")
    s = lax.axis_index("s")
    st = pl.ds(s * _ZCH, _ZCH)
    pltpu.sync_copy(zeros.at[st], stage)
    for j in range(6):
        pltpu.sync_copy(stage, sha[j].at[st])
    plsc.subcore_barrier()
    base = (s * 2 + c) * _EPW

    def step(t, carry):
        off = base + t * _CH
        pltpu.sync_copy(recv.at[pl.ds(off, _CH)], idx_v)
        for j in range(6):
            pltpu.sync_copy(ucols[j].at[pl.ds(off, _CH)], val_v)
            pltpu.sync_copy(val_v, sha[j].at[idx_v], add=True)
        return carry

    lax.fori_loop(0, _EPW // _CH, step, 0)
    plsc.subcore_barrier()
    for j in range(6):
        pltpu.sync_copy(sha[j].at[st], stage)

        @pl.when(c == 0)
        def _():
            pltpu.sync_copy(stage, out_a[j].at[st])

        @pl.when(c == 1)
        def _():
            pltpu.sync_copy(stage, out_b[j].at[st])


def _sc_scatter(ucols, recv, zeros):
    f = pl.kernel(
        _scatter_body,
        out_type=[jax.ShapeDtypeStruct((NP,), jnp.float32)] * 12,
        mesh=plsc.VectorSubcoreMesh(core_axis_name="c", subcore_axis_name="s"),
        scratch_types=(
            [pltpu.VMEM_SHARED((NP,), jnp.float32)] * 6
            + [pltpu.VMEM((_CH,), jnp.int32),
               pltpu.VMEM((_CH,), jnp.float32),
               pltpu.VMEM((_ZCH,), jnp.float32)]),
    )
    return f(*ucols, recv, zeros)


def _node_body(*refs):
    tbl_ref = refs[0]
    agg_a = refs[1:7]
    agg_b = refs[7:13]
    (wn0, bn0, wn1, bn1, wn2, bn2,
     wg0, bg0, wg1, bg1, wg2, bg2) = refs[13:25]
    out = refs[25]

    i = pl.program_id(0)

    @pl.when(i == 0)
    def _():
        out[...] = jnp.zeros_like(out)

    agg6 = _rows([agg_a[j][...] + agg_b[j][...] for j in range(6)])
    n6 = tbl_ref[0:6, :]
    nf = jnp.concatenate([agg6, n6], axis=0)
    h = _selu(_dot(wn0[...], nf) + bn0[...])
    h = _selu(_dot(wn1[...], h) + bn1[...])
    un = _dot(wn2[...], h) + bn2[...]

    lane = lax.broadcasted_iota(jnp.int32, (1, un.shape[1]), 1) + i * _BNT
    un = jnp.where(lane < N, un, 0.0)
    se = jnp.sum(agg6, axis=1, keepdims=True)
    sn = jnp.sum(un, axis=1, keepdims=True)
    out[0:6, 0:1] += se
    out[0:6, 1:2] += sn

    @pl.when(i == pl.num_programs(0) - 1)
    def _():
        g = jnp.concatenate([out[0:6, 0:1], out[0:6, 1:2]], axis=0)
        hg = _selu(_dot(wg0[...], g) + bg0[...])
        hg = _selu(_dot(wg1[...], hg) + bg1[...])
        ug = _dot(wg2[...], hg) + bg2[...]
        out[0:9, 2:3] = ug


def _tc_node(tbl_t, agg12, flat_w):
    specs = [pl.BlockSpec((8, _BNT), lambda i: (0, i))]
    specs += [pl.BlockSpec((_BNT,), lambda i: (i,))] * 12
    for a in flat_w:
        specs.append(_full(a.shape))
    return pl.pallas_call(
        _node_body,
        grid=(NP // _BNT,),
        in_specs=specs,
        out_specs=pl.BlockSpec((16, 128), lambda i: (0, 0)),
        out_shape=jax.ShapeDtypeStruct((16, 128), jnp.float32),
    )(tbl_t, *agg12, *flat_w)


def _flat_t(ps):
    out = []
    for w, b in ps:
        out.append(w.T)
        out.append(b.reshape(-1, 1))
    return out


@jax.jit
def kernel(nodes, edges, params, senders, receivers):
    recv = receivers.astype(jnp.int32)
    send = senders.astype(jnp.int32)

    nodes_t = jnp.zeros((3, NP), jnp.float32).at[:, :N].set(nodes.T)
    edges_t = edges.T

    prep_w = _flat_t(params['pn'])
    for w, b in params['pr']:
        prep_w.append(w)
        prep_w.append(b.reshape(1, -1))
    tbl_t, decoded = _tc_prep(nodes_t, nodes, prep_w)

    cols = [tbl_t[j] for j in range(6)]
    g12 = _sc_gather(cols, recv, send)

    edge_w = _flat_t(params['pe']) + _flat_t(params['em'])
    ucols = _tc_edge(edges_t, list(g12), edge_w)

    zeros = jnp.zeros((NP,), jnp.float32)
    agg12 = _sc_scatter(list(ucols), recv, zeros)

    node_w = _flat_t(params['nm']) + _flat_t(params['gm'])
    res = _tc_node(tbl_t, list(agg12), node_w)

    return decoded, res[0:9, 2]


# speedup vs baseline: 1.4644x; 1.4644x over previous
import jax
import jax.numpy as jnp
from jax import lax
from jax.experimental import pallas as pl
from jax.experimental.pallas import tpu as pltpu
from jax.experimental.pallas import tpu_sc as plsc

N = 100000
NP = 102400
E = 3200000

_M = 5
_EC = E // _M
_NW = 32
_EPW = _EC // _NW
_CH = 5000
_ZCH = NP // 16

_BNT = 5120
_BET = 25600

_SCALE = 1.0507009873554805
_ALPHA = 1.6732632423543772


def _selu(x):
    return _SCALE * jnp.where(x > 0, x, _ALPHA * (jnp.exp(x) - 1.0))


def _dot(w, x):
    return jnp.dot(w, x, preferred_element_type=jnp.float32)


def _full(shape):
    idx = tuple(0 for _ in shape)
    return pl.BlockSpec(shape, lambda *_, _idx=idx: _idx)


def _rows(mats):
    return jnp.concatenate([m.reshape(1, -1) for m in mats], axis=0)


def _prep_body(x_ref,
               wp0, bp0, wp1, bp1, wp2, bp2, wp3, bp3,
               wr0, br0, wr1, br1, wr2, br2, wr3, br3,
               tbl_ref, dec_ref):
    x = x_ref[...]
    h = _selu(_dot(wp0[...], x) + bp0[...])
    h = _selu(_dot(wp1[...], h) + bp1[...])
    h = _selu(_dot(wp2[...], h) + bp2[...])
    ln = _dot(wp3[...], h) + bp3[...]
    z = jnp.zeros((2, x.shape[1]), jnp.float32)
    tbl_ref[...] = jnp.concatenate([ln, x, z], axis=0)
    h = _selu(_dot(wr0[...], x) + br0[...])
    h = _selu(_dot(wr1[...], h) + br1[...])
    h = _selu(_dot(wr2[...], h) + br2[...])
    dec_ref[...] = _dot(wr3[...], h) + br3[...]


def _tc_prep(nodes_t, flat_w):
    specs = [pl.BlockSpec((3, _BNT), lambda i: (0, i))]
    for a in flat_w:
        specs.append(_full(a.shape))
    return pl.pallas_call(
        _prep_body,
        grid=(NP // _BNT,),
        in_specs=specs,
        out_specs=[pl.BlockSpec((8, _BNT), lambda i: (0, i)),
                   pl.BlockSpec((10, _BNT), lambda i: (0, i))],
        out_shape=[jax.ShapeDtypeStruct((8, NP), jnp.float32),
                   jax.ShapeDtypeStruct((10, NP), jnp.float32)],
    )(nodes_t, *flat_w)


def _gather_body(*refs):
    cols = refs[0:6]
    recv = refs[6]
    send = refs[7]
    out_r = refs[8:14]
    out_s = refs[14:20]
    shc = refs[20:26]
    idx_r = refs[26]
    idx_s = refs[27]
    v_r = refs[28:34]
    v_s = refs[34:40]
    stage = refs[40]
    sem = refs[41]

    c = lax.axis_index("c")
    s = lax.axis_index("s")
    st = pl.ds(s * _ZCH, _ZCH)
    for j in range(6):
        pltpu.sync_copy(cols[j].at[st], stage)
        pltpu.sync_copy(stage, shc[j].at[st])
    plsc.subcore_barrier()
    base = (s * 2 + c) * _EPW

    def step(t, carry):
        off = base + t * _CH
        pltpu.sync_copy(recv.at[pl.ds(off, _CH)], idx_r)
        pltpu.sync_copy(send.at[pl.ds(off, _CH)], idx_s)
        cps = []
        for j in range(6):
            cps.append(pltpu.async_copy(shc[j].at[idx_r], v_r[j], sem))
            cps.append(pltpu.async_copy(shc[j].at[idx_s], v_s[j], sem))
        for cp in cps:
            cp.wait()
        for j in range(6):
            pltpu.sync_copy(v_r[j], out_r[j].at[pl.ds(off, _CH)])
            pltpu.sync_copy(v_s[j], out_s[j].at[pl.ds(off, _CH)])
        return carry

    lax.fori_loop(0, _EPW // _CH, step, 0)


def _sc_gather(cols, recv, send):
    f = pl.kernel(
        _gather_body,
        out_type=[jax.ShapeDtypeStruct((_EC,), jnp.float32)] * 12,
        mesh=plsc.VectorSubcoreMesh(core_axis_name="c", subcore_axis_name="s"),
        scratch_types=(
            [pltpu.VMEM_SHARED((NP,), jnp.float32)] * 6
            + [pltpu.VMEM((_CH,), jnp.int32)] * 2
            + [pltpu.VMEM((_CH,), jnp.float32)] * 12
            + [pltpu.VMEM((_ZCH,), jnp.float32),
               pltpu.SemaphoreType.DMA]),
    )
    return f(*cols, recv, send)


def _edge_body(*refs):
    x_ref = refs[0]
    r = refs[1:7]
    s = refs[7:13]
    (we0, be0, we1, be1, we2, be2, we3, be3,
     w1, b1, w2, b2, w3, b3) = refs[13:27]
    outs = refs[27:33]

    x = x_ref[...]
    h = _selu(_dot(we0[...], x) + be0[...])
    h = _selu(_dot(we1[...], h) + be1[...])
    h = _selu(_dot(we2[...], h) + be2[...])
    pe3 = _dot(we3[...], h) + be3[...]
    r6 = _rows([q[...] for q in r])
    s6 = _rows([q[...] for q in s])
    ef = jnp.concatenate([pe3, x, r6, s6], axis=0)
    h1 = _selu(_dot(w1[...], ef) + b1[...])
    h2 = _selu(_dot(w2[...], h1) + b2[...])
    ut = _dot(w3[...], h2) + b3[...]
    for j in range(6):
        outs[j][...] = ut[j, :]


def _tc_edge(edges_t, g12, flat_w):
    specs = [pl.BlockSpec((3, _BET), lambda i: (0, i))]
    specs += [pl.BlockSpec((_BET,), lambda i: (i,))] * 12
    for a in flat_w:
        specs.append(_full(a.shape))
    return pl.pallas_call(
        _edge_body,
        grid=(_EC // _BET,),
        in_specs=specs,
        out_specs=[pl.BlockSpec((_BET,), lambda i: (i,))] * 6,
        out_shape=[jax.ShapeDtypeStruct((_EC,), jnp.float32)] * 6,
    )(edges_t, *g12, *flat_w)


def _scatter_body(*refs):
    ucols = refs[0:6]
    recv = refs[6]
    zeros = refs[7]
    out_a = refs[8:14]
    out_b = refs[14:20]
    sha = refs[20:26]
    idx_v = refs[26]
    val_v = refs[27]
    stage = refs[28]

    c = lax.axis_index("c")
    s = lax.axis_index("s")
    st = pl.ds(s * _ZCH, _ZCH)
    pltpu.sync_copy(zeros.at[st], stage)
    for j in range(6):
        pltpu.sync_copy(stage, sha[j].at[st])
    plsc.subcore_barrier()
    base = (s * 2 + c) * _EPW

    def step(t, carry):
        off = base + t * _CH
        pltpu.sync_copy(recv.at[pl.ds(off, _CH)], idx_v)
        for j in range(6):
            pltpu.sync_copy(ucols[j].at[pl.ds(off, _CH)], val_v)
            pltpu.sync_copy(val_v, sha[j].at[idx_v], add=True)
        return carry

    lax.fori_loop(0, _EPW // _CH, step, 0)
    plsc.subcore_barrier()
    for j in range(6):
        pltpu.sync_copy(sha[j].at[st], stage)

        @pl.when(c == 0)
        def _():
            pltpu.sync_copy(stage, out_a[j].at[st])

        @pl.when(c == 1)
        def _():
            pltpu.sync_copy(stage, out_b[j].at[st])


def _sc_scatter(ucols, recv, zeros):
    f = pl.kernel(
        _scatter_body,
        out_type=[jax.ShapeDtypeStruct((NP,), jnp.float32)] * 12,
        mesh=plsc.VectorSubcoreMesh(core_axis_name="c", subcore_axis_name="s"),
        scratch_types=(
            [pltpu.VMEM_SHARED((NP,), jnp.float32)] * 6
            + [pltpu.VMEM((_CH,), jnp.int32),
               pltpu.VMEM((_CH,), jnp.float32),
               pltpu.VMEM((_ZCH,), jnp.float32)]),
    )
    return f(*ucols, recv, zeros)


def _node_body(*refs):
    tbl_ref = refs[0]
    aggs = refs[1:1 + 12 * _M]
    (wn0, bn0, wn1, bn1, wn2, bn2,
     wg0, bg0, wg1, bg1, wg2, bg2) = refs[1 + 12 * _M:13 + 12 * _M]
    out = refs[13 + 12 * _M]

    i = pl.program_id(0)

    @pl.when(i == 0)
    def _():
        out[...] = jnp.zeros_like(out)

    parts = []
    for j in range(6):
        acc = aggs[j][...]
        for m in range(1, 2 * _M):
            acc = acc + aggs[6 * m + j][...]
        parts.append(acc)
    agg6 = _rows(parts)
    n6 = tbl_ref[0:6, :]
    nf = jnp.concatenate([agg6, n6], axis=0)
    h = _selu(_dot(wn0[...], nf) + bn0[...])
    h = _selu(_dot(wn1[...], h) + bn1[...])
    un = _dot(wn2[...], h) + bn2[...]

    lane = lax.broadcasted_iota(jnp.int32, (1, un.shape[1]), 1) + i * _BNT
    un = jnp.where(lane < N, un, 0.0)
    se = jnp.sum(agg6, axis=1, keepdims=True)
    sn = jnp.sum(un, axis=1, keepdims=True)
    out[0:6, 0:1] += se
    out[0:6, 1:2] += sn

    @pl.when(i == pl.num_programs(0) - 1)
    def _():
        g = jnp.concatenate([out[0:6, 0:1], out[0:6, 1:2]], axis=0)
        hg = _selu(_dot(wg0[...], g) + bg0[...])
        hg = _selu(_dot(wg1[...], hg) + bg1[...])
        ug = _dot(wg2[...], hg) + bg2[...]
        out[0:9, 2:3] = ug


def _tc_node(tbl_t, agg_all, flat_w):
    specs = [pl.BlockSpec((8, _BNT), lambda i: (0, i))]
    specs += [pl.BlockSpec((_BNT,), lambda i: (i,))] * (12 * _M)
    for a in flat_w:
        specs.append(_full(a.shape))
    return pl.pallas_call(
        _node_body,
        grid=(NP // _BNT,),
        in_specs=specs,
        out_specs=pl.BlockSpec((16, 128), lambda i: (0, 0)),
        out_shape=jax.ShapeDtypeStruct((16, 128), jnp.float32),
    )(tbl_t, *agg_all, *flat_w)


def _flat_t(ps):
    out = []
    for w, b in ps:
        out.append(w.T)
        out.append(b.reshape(-1, 1))
    return out


@jax.jit
def kernel(nodes, edges, params, senders, receivers):
    recv = receivers.astype(jnp.int32)
    send = senders.astype(jnp.int32)

    nodes_t = jnp.zeros((3, NP), jnp.float32).at[:, :N].set(nodes.T)
    edges_t = edges.T

    prep_w = _flat_t(params['pn']) + _flat_t(params['pr'])
    tbl_t, dec_t = _tc_prep(nodes_t, prep_w)

    cols = [tbl_t[j] for j in range(6)]
    edge_w = _flat_t(params['pe']) + _flat_t(params['em'])
    zeros = jnp.zeros((NP,), jnp.float32)

    agg_all = []
    for m in range(_M):
        sl = slice(m * _EC, (m + 1) * _EC)
        g12 = _sc_gather(cols, recv[sl], send[sl])
        ucols = _tc_edge(edges_t[:, sl], list(g12), edge_w)
        agg_all.extend(_sc_scatter(list(ucols), recv[sl], zeros))

    node_w = _flat_t(params['nm']) + _flat_t(params['gm'])
    res = _tc_node(tbl_t, agg_all, node_w)

    decoded = dec_t[:, :N].T
    return decoded, res[0:9, 2]


# speedup vs baseline: 1.6037x; 1.0951x over previous
import jax
import jax.numpy as jnp
from jax import lax
from jax.experimental import pallas as pl
from jax.experimental.pallas import tpu as pltpu
from jax.experimental.pallas import tpu_sc as plsc

N = 100000
NP = 102400
E = 3200000

_M = 5
_EC = E // _M
_NW = 32
_EPW = _EC // _NW
_CH = 5000
_ZCH = NP // 16

_BNT = 5120
_BET = 25600

_SCALE = 1.0507009873554805
_ALPHA = 1.6732632423543772


def _selu(x):
    return _SCALE * jnp.where(x > 0, x, _ALPHA * (jnp.exp(x) - 1.0))


def _dot(w, x):
    return jnp.dot(w, x, preferred_element_type=jnp.float32)


def _full(shape):
    idx = tuple(0 for _ in shape)
    return pl.BlockSpec(shape, lambda *_, _idx=idx: _idx)


def _rows(mats):
    return jnp.concatenate([m.reshape(1, -1) for m in mats], axis=0)


def _prep_body(x_ref,
               wp0, bp0, wp1, bp1, wp2, bp2, wp3, bp3,
               wr0, br0, wr1, br1, wr2, br2, wr3, br3,
               tbl_ref, pk_ref, dec_ref):
    x = x_ref[...]
    h = _selu(_dot(wp0[...], x) + bp0[...])
    h = _selu(_dot(wp1[...], h) + bp1[...])
    h = _selu(_dot(wp2[...], h) + bp2[...])
    ln = _dot(wp3[...], h) + bp3[...]
    z = jnp.zeros((2, x.shape[1]), jnp.float32)
    f6 = jnp.concatenate([ln, x], axis=0)
    tbl_ref[...] = jnp.concatenate([f6, z], axis=0)
    bf = f6.astype(jnp.bfloat16)
    u = lax.bitcast_convert_type(bf, jnp.uint16).astype(jnp.uint32)
    rows = []
    for p in range(3):
        rows.append((u[2 * p + 1:2 * p + 2, :] << 16) | u[2 * p:2 * p + 1, :])
    pk_ref[...] = lax.bitcast_convert_type(
        jnp.concatenate(rows, axis=0), jnp.int32)
    h = _selu(_dot(wr0[...], x) + br0[...])
    h = _selu(_dot(wr1[...], h) + br1[...])
    h = _selu(_dot(wr2[...], h) + br2[...])
    dec_ref[...] = _dot(wr3[...], h) + br3[...]


def _tc_prep(nodes_t, flat_w):
    specs = [pl.BlockSpec((3, _BNT), lambda i: (0, i))]
    for a in flat_w:
        specs.append(_full(a.shape))
    return pl.pallas_call(
        _prep_body,
        grid=(NP // _BNT,),
        in_specs=specs,
        out_specs=[pl.BlockSpec((8, _BNT), lambda i: (0, i)),
                   pl.BlockSpec((3, _BNT), lambda i: (0, i)),
                   pl.BlockSpec((10, _BNT), lambda i: (0, i))],
        out_shape=[jax.ShapeDtypeStruct((8, NP), jnp.float32),
                   jax.ShapeDtypeStruct((3, NP), jnp.int32),
                   jax.ShapeDtypeStruct((10, NP), jnp.float32)],
    )(nodes_t, *flat_w)


def _gather_body(*refs):
    cols = refs[0:3]
    recv = refs[3]
    send = refs[4]
    out_r = refs[5:8]
    out_s = refs[8:11]
    shc = refs[11:14]
    idx_r = refs[14]
    idx_s = refs[15]
    v_r = refs[16:19]
    v_s = refs[19:22]
    stage = refs[22]
    sem = refs[23]

    c = lax.axis_index("c")
    s = lax.axis_index("s")
    st = pl.ds(s * _ZCH, _ZCH)
    for j in range(3):
        pltpu.sync_copy(cols[j].at[st], stage)
        pltpu.sync_copy(stage, shc[j].at[st])
    plsc.subcore_barrier()
    base = (s * 2 + c) * _EPW

    def step(t, carry):
        off = base + t * _CH
        pltpu.sync_copy(recv.at[pl.ds(off, _CH)], idx_r)
        pltpu.sync_copy(send.at[pl.ds(off, _CH)], idx_s)
        cps = []
        for j in range(3):
            cps.append(pltpu.async_copy(shc[j].at[idx_r], v_r[j], sem))
            cps.append(pltpu.async_copy(shc[j].at[idx_s], v_s[j], sem))
        for cp in cps:
            cp.wait()
        for j in range(3):
            pltpu.sync_copy(v_r[j], out_r[j].at[pl.ds(off, _CH)])
            pltpu.sync_copy(v_s[j], out_s[j].at[pl.ds(off, _CH)])
        return carry

    lax.fori_loop(0, _EPW // _CH, step, 0)


def _sc_gather(cols, recv, send):
    f = pl.kernel(
        _gather_body,
        out_type=[jax.ShapeDtypeStruct((_EC,), jnp.int32)] * 6,
        mesh=plsc.VectorSubcoreMesh(core_axis_name="c", subcore_axis_name="s"),
        scratch_types=(
            [pltpu.VMEM_SHARED((NP,), jnp.int32)] * 3
            + [pltpu.VMEM((_CH,), jnp.int32)] * 2
            + [pltpu.VMEM((_CH,), jnp.int32)] * 6
            + [pltpu.VMEM((_ZCH,), jnp.int32),
               pltpu.SemaphoreType.DMA]),
    )
    return f(*cols, recv, send)


def _unpack6(packed):
    rows = []
    for p in range(3):
        u = lax.bitcast_convert_type(packed[p][...].reshape(1, -1),
                                     jnp.uint32)
        lo = lax.bitcast_convert_type(u.astype(jnp.uint16), jnp.bfloat16)
        hi = lax.bitcast_convert_type((u >> 16).astype(jnp.uint16),
                                      jnp.bfloat16)
        rows.append(lo.astype(jnp.float32))
        rows.append(hi.astype(jnp.float32))
    return jnp.concatenate(rows, axis=0)


def _edge_body(*refs):
    x_ref = refs[0]
    r = refs[1:4]
    s = refs[4:7]
    (we0, be0, we1, be1, we2, be2, we3, be3,
     w1, b1, w2, b2, w3, b3) = refs[7:21]
    outs = refs[21:27]

    x = x_ref[...]
    h = _selu(_dot(we0[...], x) + be0[...])
    h = _selu(_dot(we1[...], h) + be1[...])
    h = _selu(_dot(we2[...], h) + be2[...])
    pe3 = _dot(we3[...], h) + be3[...]
    r6 = _unpack6(r)
    s6 = _unpack6(s)
    ef = jnp.concatenate([pe3, x, r6, s6], axis=0)
    h1 = _selu(_dot(w1[...], ef) + b1[...])
    h2 = _selu(_dot(w2[...], h1) + b2[...])
    ut = _dot(w3[...], h2) + b3[...]
    for j in range(6):
        outs[j][...] = ut[j, :]


def _tc_edge(edges_t, g6, flat_w):
    specs = [pl.BlockSpec((3, _BET), lambda i: (0, i))]
    specs += [pl.BlockSpec((_BET,), lambda i: (i,))] * 6
    for a in flat_w:
        specs.append(_full(a.shape))
    return pl.pallas_call(
        _edge_body,
        grid=(_EC // _BET,),
        in_specs=specs,
        out_specs=[pl.BlockSpec((_BET,), lambda i: (i,))] * 6,
        out_shape=[jax.ShapeDtypeStruct((_EC,), jnp.float32)] * 6,
    )(edges_t, *g6, *flat_w)


def _scatter_body(*refs):
    ucols = refs[0:6]
    recv = refs[6]
    zeros = refs[7]
    out_a = refs[8:14]
    out_b = refs[14:20]
    sha = refs[20:26]
    idx_v = refs[26]
    val_v = refs[27]
    stage = refs[28]

    c = lax.axis_index("c")
    s = lax.axis_index("s")
    st = pl.ds(s * _ZCH, _ZCH)
    pltpu.sync_copy(zeros.at[st], stage)
    for j in range(6):
        pltpu.sync_copy(stage, sha[j].at[st])
    plsc.subcore_barrier()
    base = (s * 2 + c) * _EPW

    def step(t, carry):
        off = base + t * _CH
        pltpu.sync_copy(recv.at[pl.ds(off, _CH)], idx_v)
        for j in range(6):
            pltpu.sync_copy(ucols[j].at[pl.ds(off, _CH)], val_v)
            pltpu.sync_copy(val_v, sha[j].at[idx_v], add=True)
        return carry

    lax.fori_loop(0, _EPW // _CH, step, 0)
    plsc.subcore_barrier()
    for j in range(6):
        pltpu.sync_copy(sha[j].at[st], stage)

        @pl.when(c == 0)
        def _():
            pltpu.sync_copy(stage, out_a[j].at[st])

        @pl.when(c == 1)
        def _():
            pltpu.sync_copy(stage, out_b[j].at[st])


def _sc_scatter(ucols, recv, zeros):
    f = pl.kernel(
        _scatter_body,
        out_type=[jax.ShapeDtypeStruct((NP,), jnp.float32)] * 12,
        mesh=plsc.VectorSubcoreMesh(core_axis_name="c", subcore_axis_name="s"),
        scratch_types=(
            [pltpu.VMEM_SHARED((NP,), jnp.float32)] * 6
            + [pltpu.VMEM((_CH,), jnp.int32),
               pltpu.VMEM((_CH,), jnp.float32),
               pltpu.VMEM((_ZCH,), jnp.float32)]),
    )
    return f(*ucols, recv, zeros)


def _node_body(*refs):
    tbl_ref = refs[0]
    aggs = refs[1:1 + 12 * _M]
    (wn0, bn0, wn1, bn1, wn2, bn2,
     wg0, bg0, wg1, bg1, wg2, bg2) = refs[1 + 12 * _M:13 + 12 * _M]
    out = refs[13 + 12 * _M]

    i = pl.program_id(0)

    @pl.when(i == 0)
    def _():
        out[...] = jnp.zeros_like(out)

    parts = []
    for j in range(6):
        acc = aggs[j][...]
        for m in range(1, 2 * _M):
            acc = acc + aggs[6 * m + j][...]
        parts.append(acc)
    agg6 = _rows(parts)
    n6 = tbl_ref[0:6, :]
    nf = jnp.concatenate([agg6, n6], axis=0)
    h = _selu(_dot(wn0[...], nf) + bn0[...])
    h = _selu(_dot(wn1[...], h) + bn1[...])
    un = _dot(wn2[...], h) + bn2[...]

    lane = lax.broadcasted_iota(jnp.int32, (1, un.shape[1]), 1) + i * _BNT
    un = jnp.where(lane < N, un, 0.0)
    se = jnp.sum(agg6, axis=1, keepdims=True)
    sn = jnp.sum(un, axis=1, keepdims=True)
    out[0:6, 0:1] += se
    out[0:6, 1:2] += sn

    @pl.when(i == pl.num_programs(0) - 1)
    def _():
        g = jnp.concatenate([out[0:6, 0:1], out[0:6, 1:2]], axis=0)
        hg = _selu(_dot(wg0[...], g) + bg0[...])
        hg = _selu(_dot(wg1[...], hg) + bg1[...])
        ug = _dot(wg2[...], hg) + bg2[...]
        out[0:9, 2:3] = ug


def _tc_node(tbl_t, agg_all, flat_w):
    specs = [pl.BlockSpec((8, _BNT), lambda i: (0, i))]
    specs += [pl.BlockSpec((_BNT,), lambda i: (i,))] * (12 * _M)
    for a in flat_w:
        specs.append(_full(a.shape))
    return pl.pallas_call(
        _node_body,
        grid=(NP // _BNT,),
        in_specs=specs,
        out_specs=pl.BlockSpec((16, 128), lambda i: (0, 0)),
        out_shape=jax.ShapeDtypeStruct((16, 128), jnp.float32),
    )(tbl_t, *agg_all, *flat_w)


def _flat_t(ps):
    out = []
    for w, b in ps:
        out.append(w.T)
        out.append(b.reshape(-1, 1))
    return out


@jax.jit
def kernel(nodes, edges, params, senders, receivers):
    recv = receivers.astype(jnp.int32)
    send = senders.astype(jnp.int32)

    nodes_t = jnp.zeros((3, NP), jnp.float32).at[:, :N].set(nodes.T)
    edges_t = edges.T

    prep_w = _flat_t(params['pn']) + _flat_t(params['pr'])
    tbl_t, pk_t, dec_t = _tc_prep(nodes_t, prep_w)

    cols = [pk_t[j] for j in range(3)]
    edge_w = _flat_t(params['pe']) + _flat_t(params['em'])
    zeros = jnp.zeros((NP,), jnp.float32)

    agg_all = []
    for m in range(_M):
        sl = slice(m * _EC, (m + 1) * _EC)
        g6 = _sc_gather(cols, recv[sl], send[sl])
        ucols = _tc_edge(edges_t[:, sl], list(g6), edge_w)
        agg_all.extend(_sc_scatter(list(ucols), recv[sl], zeros))

    node_w = _flat_t(params['nm']) + _flat_t(params['gm'])
    res = _tc_node(tbl_t, agg_all, node_w)

    decoded = dec_t[:, :N].T
    return decoded, res[0:9, 2]


# speedup vs baseline: 1.6280x; 1.0152x over previous
import jax
import jax.numpy as jnp
from jax import lax
from jax.experimental import pallas as pl
from jax.experimental.pallas import tpu as pltpu
from jax.experimental.pallas import tpu_sc as plsc

N = 100000
NP = 102400
E = 3200000

_M = 5
_EC = E // _M
_NW = 32
_EPW = _EC // _NW
_CH = 5000
_ZCH = NP // 16

_BNT = 5120
_BET = 25600

_SCALE = 1.0507009873554805
_ALPHA = 1.6732632423543772


def _selu(x):
    return _SCALE * jnp.where(x > 0, x, _ALPHA * (jnp.exp(x) - 1.0))


def _dot(w, x):
    return jnp.dot(w, x, preferred_element_type=jnp.float32)


def _full(shape):
    idx = tuple(0 for _ in shape)
    return pl.BlockSpec(shape, lambda *_, _idx=idx: _idx)


def _rows(mats):
    return jnp.concatenate([m.reshape(1, -1) for m in mats], axis=0)


def _prep_body(x_ref,
               wp0, bp0, wp1, bp1, wp2, bp2, wp3, bp3,
               wr0, br0, wr1, br1, wr2, br2, wr3, br3,
               tbl_ref, pk_ref, dec_ref):
    x = x_ref[...]
    h = _selu(_dot(wp0[...], x) + bp0[...])
    h = _selu(_dot(wp1[...], h) + bp1[...])
    h = _selu(_dot(wp2[...], h) + bp2[...])
    ln = _dot(wp3[...], h) + bp3[...]
    z = jnp.zeros((2, x.shape[1]), jnp.float32)
    f6 = jnp.concatenate([ln, x], axis=0)
    tbl_ref[...] = jnp.concatenate([f6, z], axis=0)
    bf = f6.astype(jnp.bfloat16)
    u = lax.bitcast_convert_type(bf, jnp.uint16).astype(jnp.uint32)
    rows = []
    for p in range(3):
        rows.append((u[2 * p + 1:2 * p + 2, :] << 16) | u[2 * p:2 * p + 1, :])
    pk_ref[...] = lax.bitcast_convert_type(
        jnp.concatenate(rows, axis=0), jnp.int32)
    h = _selu(_dot(wr0[...], x) + br0[...])
    h = _selu(_dot(wr1[...], h) + br1[...])
    h = _selu(_dot(wr2[...], h) + br2[...])
    dec_ref[...] = _dot(wr3[...], h) + br3[...]


def _tc_prep(nodes_t, flat_w):
    specs = [pl.BlockSpec((3, _BNT), lambda i: (0, i))]
    for a in flat_w:
        specs.append(_full(a.shape))
    return pl.pallas_call(
        _prep_body,
        grid=(NP // _BNT,),
        in_specs=specs,
        out_specs=[pl.BlockSpec((8, _BNT), lambda i: (0, i)),
                   pl.BlockSpec((3, _BNT), lambda i: (0, i)),
                   pl.BlockSpec((10, _BNT), lambda i: (0, i))],
        out_shape=[jax.ShapeDtypeStruct((8, NP), jnp.float32),
                   jax.ShapeDtypeStruct((3, NP), jnp.int32),
                   jax.ShapeDtypeStruct((10, NP), jnp.float32)],
    )(nodes_t, *flat_w)


def _gather_body(*refs):
    cols = refs[0:3]
    recv = refs[3]
    send = refs[4]
    out_r = refs[5:8]
    out_s = refs[8:11]
    shc = refs[11:14]
    idx_r = refs[14]
    idx_s = refs[15]
    v_r = refs[16:19]
    v_s = refs[19:22]
    stage = refs[22]
    sem = refs[23]

    c = lax.axis_index("c")
    s = lax.axis_index("s")
    st = pl.ds(s * _ZCH, _ZCH)
    for j in range(3):
        pltpu.sync_copy(cols[j].at[st], stage)
        pltpu.sync_copy(stage, shc[j].at[st])
    plsc.subcore_barrier()
    base = (s * 2 + c) * _EPW

    def step(t, carry):
        off = base + t * _CH
        pltpu.sync_copy(recv.at[pl.ds(off, _CH)], idx_r)
        pltpu.sync_copy(send.at[pl.ds(off, _CH)], idx_s)
        cps = []
        for j in range(3):
            cps.append(pltpu.async_copy(shc[j].at[idx_r], v_r[j], sem))
            cps.append(pltpu.async_copy(shc[j].at[idx_s], v_s[j], sem))
        for cp in cps:
            cp.wait()
        for j in range(3):
            pltpu.sync_copy(v_r[j], out_r[j].at[pl.ds(off, _CH)])
            pltpu.sync_copy(v_s[j], out_s[j].at[pl.ds(off, _CH)])
        return carry

    lax.fori_loop(0, _EPW // _CH, step, 0)


def _sc_gather(cols, recv, send):
    f = pl.kernel(
        _gather_body,
        out_type=[jax.ShapeDtypeStruct((_EC,), jnp.int32)] * 6,
        mesh=plsc.VectorSubcoreMesh(core_axis_name="c", subcore_axis_name="s"),
        scratch_types=(
            [pltpu.VMEM_SHARED((NP,), jnp.int32)] * 3
            + [pltpu.VMEM((_CH,), jnp.int32)] * 2
            + [pltpu.VMEM((_CH,), jnp.int32)] * 6
            + [pltpu.VMEM((_ZCH,), jnp.int32),
               pltpu.SemaphoreType.DMA]),
    )
    return f(*cols, recv, send)


def _unpack6(packed):
    rows = []
    for p in range(3):
        u = lax.bitcast_convert_type(packed[p][...].reshape(1, -1),
                                     jnp.uint32)
        lo = lax.bitcast_convert_type(u.astype(jnp.uint16), jnp.bfloat16)
        hi = lax.bitcast_convert_type((u >> 16).astype(jnp.uint16),
                                      jnp.bfloat16)
        rows.append(lo.astype(jnp.float32))
        rows.append(hi.astype(jnp.float32))
    return jnp.concatenate(rows, axis=0)


def _edge_body(*refs):
    x_ref = refs[0]
    r = refs[1:4]
    s = refs[4:7]
    (we0, be0, we1, be1, we2, be2, we3, be3,
     w1, b1, w2, b2, w3, b3) = refs[7:21]
    outs = refs[21:27]

    x = x_ref[...]
    h = _selu(_dot(we0[...], x) + be0[...])
    h = _selu(_dot(we1[...], h) + be1[...])
    h = _selu(_dot(we2[...], h) + be2[...])
    pe3 = _dot(we3[...], h) + be3[...]
    r6 = _unpack6(r)
    s6 = _unpack6(s)
    ef = jnp.concatenate([pe3, x, r6, s6], axis=0)
    h1 = _selu(_dot(w1[...], ef) + b1[...])
    h2 = _selu(_dot(w2[...], h1) + b2[...])
    ut = _dot(w3[...], h2) + b3[...]
    for j in range(6):
        outs[j][...] = ut[j, :]


def _tc_edge(edges_t, g6, flat_w):
    specs = [pl.BlockSpec((3, _BET), lambda i: (0, i))]
    specs += [pl.BlockSpec((_BET,), lambda i: (i,))] * 6
    for a in flat_w:
        specs.append(_full(a.shape))
    return pl.pallas_call(
        _edge_body,
        grid=(_EC // _BET,),
        in_specs=specs,
        out_specs=[pl.BlockSpec((_BET,), lambda i: (i,))] * 6,
        out_shape=[jax.ShapeDtypeStruct((_EC,), jnp.float32)] * 6,
    )(edges_t, *g6, *flat_w)


def _scatter_body(*refs):
    ucols = refs[0:6]
    recv = refs[6]
    zeros = refs[7]
    out_a = refs[8:14]
    out_b = refs[14:20]
    sha = refs[20:26]
    idx_v = refs[26]
    vals = refs[27:33]
    stage = refs[33]
    sem = refs[34]

    c = lax.axis_index("c")
    s = lax.axis_index("s")
    st = pl.ds(s * _ZCH, _ZCH)
    pltpu.sync_copy(zeros.at[st], stage)
    for j in range(6):
        pltpu.sync_copy(stage, sha[j].at[st])
    plsc.subcore_barrier()
    base = (s * 2 + c) * _EPW

    def step(t, carry):
        off = base + t * _CH
        pltpu.sync_copy(recv.at[pl.ds(off, _CH)], idx_v)
        cps = [pltpu.async_copy(ucols[j].at[pl.ds(off, _CH)], vals[j], sem)
               for j in range(6)]
        for cp in cps:
            cp.wait()
        for j in range(6):
            pltpu.sync_copy(vals[j], sha[j].at[idx_v], add=True)
        return carry

    lax.fori_loop(0, _EPW // _CH, step, 0)
    plsc.subcore_barrier()
    for j in range(6):
        pltpu.sync_copy(sha[j].at[st], stage)

        @pl.when(c == 0)
        def _():
            pltpu.sync_copy(stage, out_a[j].at[st])

        @pl.when(c == 1)
        def _():
            pltpu.sync_copy(stage, out_b[j].at[st])


def _sc_scatter(ucols, recv, zeros):
    f = pl.kernel(
        _scatter_body,
        out_type=[jax.ShapeDtypeStruct((NP,), jnp.float32)] * 12,
        mesh=plsc.VectorSubcoreMesh(core_axis_name="c", subcore_axis_name="s"),
        scratch_types=(
            [pltpu.VMEM_SHARED((NP,), jnp.float32)] * 6
            + [pltpu.VMEM((_CH,), jnp.int32)]
            + [pltpu.VMEM((_CH,), jnp.float32)] * 6
            + [pltpu.VMEM((_ZCH,), jnp.float32),
               pltpu.SemaphoreType.DMA]),
    )
    return f(*ucols, recv, zeros)


def _node_body(*refs):
    tbl_ref = refs[0]
    aggs = refs[1:1 + 12 * _M]
    (wn0, bn0, wn1, bn1, wn2, bn2,
     wg0, bg0, wg1, bg1, wg2, bg2) = refs[1 + 12 * _M:13 + 12 * _M]
    out = refs[13 + 12 * _M]

    i = pl.program_id(0)

    @pl.when(i == 0)
    def _():
        out[...] = jnp.zeros_like(out)

    parts = []
    for j in range(6):
        acc = aggs[j][...]
        for m in range(1, 2 * _M):
            acc = acc + aggs[6 * m + j][...]
        parts.append(acc)
    agg6 = _rows(parts)
    n6 = tbl_ref[0:6, :]
    nf = jnp.concatenate([agg6, n6], axis=0)
    h = _selu(_dot(wn0[...], nf) + bn0[...])
    h = _selu(_dot(wn1[...], h) + bn1[...])
    un = _dot(wn2[...], h) + bn2[...]

    lane = lax.broadcasted_iota(jnp.int32, (1, un.shape[1]), 1) + i * _BNT
    un = jnp.where(lane < N, un, 0.0)
    se = jnp.sum(agg6, axis=1, keepdims=True)
    sn = jnp.sum(un, axis=1, keepdims=True)
    out[0:6, 0:1] += se
    out[0:6, 1:2] += sn

    @pl.when(i == pl.num_programs(0) - 1)
    def _():
        g = jnp.concatenate([out[0:6, 0:1], out[0:6, 1:2]], axis=0)
        hg = _selu(_dot(wg0[...], g) + bg0[...])
        hg = _selu(_dot(wg1[...], hg) + bg1[...])
        ug = _dot(wg2[...], hg) + bg2[...]
        out[0:9, 2:3] = ug


def _tc_node(tbl_t, agg_all, flat_w):
    specs = [pl.BlockSpec((8, _BNT), lambda i: (0, i))]
    specs += [pl.BlockSpec((_BNT,), lambda i: (i,))] * (12 * _M)
    for a in flat_w:
        specs.append(_full(a.shape))
    return pl.pallas_call(
        _node_body,
        grid=(NP // _BNT,),
        in_specs=specs,
        out_specs=pl.BlockSpec((16, 128), lambda i: (0, 0)),
        out_shape=jax.ShapeDtypeStruct((16, 128), jnp.float32),
    )(tbl_t, *agg_all, *flat_w)


def _flat_t(ps):
    out = []
    for w, b in ps:
        out.append(w.T)
        out.append(b.reshape(-1, 1))
    return out


@jax.jit
def kernel(nodes, edges, params, senders, receivers):
    recv = receivers.astype(jnp.int32)
    send = senders.astype(jnp.int32)

    nodes_t = jnp.zeros((3, NP), jnp.float32).at[:, :N].set(nodes.T)
    edges_t = edges.T

    prep_w = _flat_t(params['pn']) + _flat_t(params['pr'])
    tbl_t, pk_t, dec_t = _tc_prep(nodes_t, prep_w)

    cols = [pk_t[j] for j in range(3)]
    edge_w = _flat_t(params['pe']) + _flat_t(params['em'])
    zeros = jnp.zeros((NP,), jnp.float32)

    agg_all = []
    for m in range(_M):
        sl = slice(m * _EC, (m + 1) * _EC)
        g6 = _sc_gather(cols, recv[sl], send[sl])
        ucols = _tc_edge(edges_t[:, sl], list(g6), edge_w)
        agg_all.extend(_sc_scatter(list(ucols), recv[sl], zeros))

    node_w = _flat_t(params['nm']) + _flat_t(params['gm'])
    res = _tc_node(tbl_t, agg_all, node_w)

    decoded = dec_t[:, :N].T
    return decoded, res[0:9, 2]


# speedup vs baseline: 1.6367x; 1.0054x over previous
import jax
import jax.numpy as jnp
from jax import lax
from jax.experimental import pallas as pl
from jax.experimental.pallas import tpu as pltpu
from jax.experimental.pallas import tpu_sc as plsc

N = 100000
NP = 102400
E = 3200000

_M = 5
_EC = E // _M
_NW = 32
_EPW = _EC // _NW
_CH = 10000
_ZCH = NP // 16

_BNT = 5120
_BET = 25600

_SCALE = 1.0507009873554805
_ALPHA = 1.6732632423543772


def _selu(x):
    return _SCALE * jnp.where(x > 0, x, _ALPHA * (jnp.exp(x) - 1.0))


def _dot(w, x):
    return jnp.dot(w, x, preferred_element_type=jnp.float32)


def _full(shape):
    idx = tuple(0 for _ in shape)
    return pl.BlockSpec(shape, lambda *_, _idx=idx: _idx)


def _rows(mats):
    return jnp.concatenate([m.reshape(1, -1) for m in mats], axis=0)


def _prep_body(x_ref,
               wp0, bp0, wp1, bp1, wp2, bp2, wp3, bp3,
               wr0, br0, wr1, br1, wr2, br2, wr3, br3,
               tbl_ref, pk_ref, dec_ref):
    x = x_ref[...]
    h = _selu(_dot(wp0[...], x) + bp0[...])
    h = _selu(_dot(wp1[...], h) + bp1[...])
    h = _selu(_dot(wp2[...], h) + bp2[...])
    ln = _dot(wp3[...], h) + bp3[...]
    z = jnp.zeros((2, x.shape[1]), jnp.float32)
    f6 = jnp.concatenate([ln, x], axis=0)
    tbl_ref[...] = jnp.concatenate([f6, z], axis=0)
    bf = f6.astype(jnp.bfloat16)
    u = lax.bitcast_convert_type(bf, jnp.uint16).astype(jnp.uint32)
    rows = []
    for p in range(3):
        rows.append((u[2 * p + 1:2 * p + 2, :] << 16) | u[2 * p:2 * p + 1, :])
    pk_ref[...] = lax.bitcast_convert_type(
        jnp.concatenate(rows, axis=0), jnp.int32)
    h = _selu(_dot(wr0[...], x) + br0[...])
    h = _selu(_dot(wr1[...], h) + br1[...])
    h = _selu(_dot(wr2[...], h) + br2[...])
    dec_ref[...] = _dot(wr3[...], h) + br3[...]


def _tc_prep(nodes_t, flat_w):
    specs = [pl.BlockSpec((3, _BNT), lambda i: (0, i))]
    for a in flat_w:
        specs.append(_full(a.shape))
    return pl.pallas_call(
        _prep_body,
        grid=(NP // _BNT,),
        in_specs=specs,
        out_specs=[pl.BlockSpec((8, _BNT), lambda i: (0, i)),
                   pl.BlockSpec((3, _BNT), lambda i: (0, i)),
                   pl.BlockSpec((10, _BNT), lambda i: (0, i))],
        out_shape=[jax.ShapeDtypeStruct((8, NP), jnp.float32),
                   jax.ShapeDtypeStruct((3, NP), jnp.int32),
                   jax.ShapeDtypeStruct((10, NP), jnp.float32)],
    )(nodes_t, *flat_w)


def _gather_body(*refs):
    cols = refs[0:3]
    recv = refs[3]
    send = refs[4]
    out_r = refs[5:8]
    out_s = refs[8:11]
    shc = refs[11:14]
    idx_r = refs[14]
    idx_s = refs[15]
    v_r = refs[16:19]
    v_s = refs[19:22]
    stage = refs[22]
    sem = refs[23]

    c = lax.axis_index("c")
    s = lax.axis_index("s")
    st = pl.ds(s * _ZCH, _ZCH)
    for j in range(3):
        pltpu.sync_copy(cols[j].at[st], stage)
        pltpu.sync_copy(stage, shc[j].at[st])
    plsc.subcore_barrier()
    base = (s * 2 + c) * _EPW

    def step(t, carry):
        off = base + t * _CH
        pltpu.sync_copy(recv.at[pl.ds(off, _CH)], idx_r)
        pltpu.sync_copy(send.at[pl.ds(off, _CH)], idx_s)
        cps = []
        for j in range(3):
            cps.append(pltpu.async_copy(shc[j].at[idx_r], v_r[j], sem))
            cps.append(pltpu.async_copy(shc[j].at[idx_s], v_s[j], sem))
        for cp in cps:
            cp.wait()
        for j in range(3):
            pltpu.sync_copy(v_r[j], out_r[j].at[pl.ds(off, _CH)])
            pltpu.sync_copy(v_s[j], out_s[j].at[pl.ds(off, _CH)])
        return carry

    lax.fori_loop(0, _EPW // _CH, step, 0)


def _sc_gather(cols, recv, send):
    f = pl.kernel(
        _gather_body,
        out_type=[jax.ShapeDtypeStruct((_EC,), jnp.int32)] * 6,
        mesh=plsc.VectorSubcoreMesh(core_axis_name="c", subcore_axis_name="s"),
        scratch_types=(
            [pltpu.VMEM_SHARED((NP,), jnp.int32)] * 3
            + [pltpu.VMEM((_CH,), jnp.int32)] * 2
            + [pltpu.VMEM((_CH,), jnp.int32)] * 6
            + [pltpu.VMEM((_ZCH,), jnp.int32),
               pltpu.SemaphoreType.DMA]),
    )
    return f(*cols, recv, send)


def _unpack6(packed):
    rows = []
    for p in range(3):
        u = lax.bitcast_convert_type(packed[p][...].reshape(1, -1),
                                     jnp.uint32)
        lo = lax.bitcast_convert_type(u.astype(jnp.uint16), jnp.bfloat16)
        hi = lax.bitcast_convert_type((u >> 16).astype(jnp.uint16),
                                      jnp.bfloat16)
        rows.append(lo.astype(jnp.float32))
        rows.append(hi.astype(jnp.float32))
    return jnp.concatenate(rows, axis=0)


def _edge_body(*refs):
    x_ref = refs[0]
    r = refs[1:4]
    s = refs[4:7]
    (we0, be0, we1, be1, we2, be2, we3, be3,
     w1, b1, w2, b2, w3, b3) = refs[7:21]
    outs = refs[21:27]

    x = x_ref[...]
    h = _selu(_dot(we0[...], x) + be0[...])
    h = _selu(_dot(we1[...], h) + be1[...])
    h = _selu(_dot(we2[...], h) + be2[...])
    pe3 = _dot(we3[...], h) + be3[...]
    r6 = _unpack6(r)
    s6 = _unpack6(s)
    ef = jnp.concatenate([pe3, x, r6, s6], axis=0)
    h1 = _selu(_dot(w1[...], ef) + b1[...])
    h2 = _selu(_dot(w2[...], h1) + b2[...])
    ut = _dot(w3[...], h2) + b3[...]
    for j in range(6):
        outs[j][...] = ut[j, :]


def _tc_edge(edges_t, g6, flat_w):
    specs = [pl.BlockSpec((3, _BET), lambda i: (0, i))]
    specs += [pl.BlockSpec((_BET,), lambda i: (i,))] * 6
    for a in flat_w:
        specs.append(_full(a.shape))
    return pl.pallas_call(
        _edge_body,
        grid=(_EC // _BET,),
        in_specs=specs,
        out_specs=[pl.BlockSpec((_BET,), lambda i: (i,))] * 6,
        out_shape=[jax.ShapeDtypeStruct((_EC,), jnp.float32)] * 6,
    )(edges_t, *g6, *flat_w)


def _scatter_body(*refs):
    ucols = refs[0:6]
    recv = refs[6]
    zeros = refs[7]
    out_a = refs[8:14]
    out_b = refs[14:20]
    sha = refs[20:26]
    idx_v = refs[26]
    vals = refs[27:33]
    stage = refs[33]
    sem = refs[34]

    c = lax.axis_index("c")
    s = lax.axis_index("s")
    st = pl.ds(s * _ZCH, _ZCH)
    pltpu.sync_copy(zeros.at[st], stage)
    for j in range(6):
        pltpu.sync_copy(stage, sha[j].at[st])
    plsc.subcore_barrier()
    base = (s * 2 + c) * _EPW

    def step(t, carry):
        off = base + t * _CH
        pltpu.sync_copy(recv.at[pl.ds(off, _CH)], idx_v)
        cps = [pltpu.async_copy(ucols[j].at[pl.ds(off, _CH)], vals[j], sem)
               for j in range(6)]
        for cp in cps:
            cp.wait()
        for j in range(6):
            pltpu.sync_copy(vals[j], sha[j].at[idx_v], add=True)
        return carry

    lax.fori_loop(0, _EPW // _CH, step, 0)
    plsc.subcore_barrier()
    for j in range(6):
        pltpu.sync_copy(sha[j].at[st], stage)

        @pl.when(c == 0)
        def _():
            pltpu.sync_copy(stage, out_a[j].at[st])

        @pl.when(c == 1)
        def _():
            pltpu.sync_copy(stage, out_b[j].at[st])


def _sc_scatter(ucols, recv, zeros):
    f = pl.kernel(
        _scatter_body,
        out_type=[jax.ShapeDtypeStruct((NP,), jnp.float32)] * 12,
        mesh=plsc.VectorSubcoreMesh(core_axis_name="c", subcore_axis_name="s"),
        scratch_types=(
            [pltpu.VMEM_SHARED((NP,), jnp.float32)] * 6
            + [pltpu.VMEM((_CH,), jnp.int32)]
            + [pltpu.VMEM((_CH,), jnp.float32)] * 6
            + [pltpu.VMEM((_ZCH,), jnp.float32),
               pltpu.SemaphoreType.DMA]),
    )
    return f(*ucols, recv, zeros)


def _node_body(*refs):
    tbl_ref = refs[0]
    aggs = refs[1:1 + 12 * _M]
    (wn0, bn0, wn1, bn1, wn2, bn2,
     wg0, bg0, wg1, bg1, wg2, bg2) = refs[1 + 12 * _M:13 + 12 * _M]
    out = refs[13 + 12 * _M]

    i = pl.program_id(0)

    @pl.when(i == 0)
    def _():
        out[...] = jnp.zeros_like(out)

    parts = []
    for j in range(6):
        acc = aggs[j][...]
        for m in range(1, 2 * _M):
            acc = acc + aggs[6 * m + j][...]
        parts.append(acc)
    agg6 = _rows(parts)
    n6 = tbl_ref[0:6, :]
    nf = jnp.concatenate([agg6, n6], axis=0)
    h = _selu(_dot(wn0[...], nf) + bn0[...])
    h = _selu(_dot(wn1[...], h) + bn1[...])
    un = _dot(wn2[...], h) + bn2[...]

    lane = lax.broadcasted_iota(jnp.int32, (1, un.shape[1]), 1) + i * _BNT
    un = jnp.where(lane < N, un, 0.0)
    se = jnp.sum(agg6, axis=1, keepdims=True)
    sn = jnp.sum(un, axis=1, keepdims=True)
    out[0:6, 0:1] += se
    out[0:6, 1:2] += sn

    @pl.when(i == pl.num_programs(0) - 1)
    def _():
        g = jnp.concatenate([out[0:6, 0:1], out[0:6, 1:2]], axis=0)
        hg = _selu(_dot(wg0[...], g) + bg0[...])
        hg = _selu(_dot(wg1[...], hg) + bg1[...])
        ug = _dot(wg2[...], hg) + bg2[...]
        out[0:9, 2:3] = ug


def _tc_node(tbl_t, agg_all, flat_w):
    specs = [pl.BlockSpec((8, _BNT), lambda i: (0, i))]
    specs += [pl.BlockSpec((_BNT,), lambda i: (i,))] * (12 * _M)
    for a in flat_w:
        specs.append(_full(a.shape))
    return pl.pallas_call(
        _node_body,
        grid=(NP // _BNT,),
        in_specs=specs,
        out_specs=pl.BlockSpec((16, 128), lambda i: (0, 0)),
        out_shape=jax.ShapeDtypeStruct((16, 128), jnp.float32),
    )(tbl_t, *agg_all, *flat_w)


def _flat_t(ps):
    out = []
    for w, b in ps:
        out.append(w.T)
        out.append(b.reshape(-1, 1))
    return out


@jax.jit
def kernel(nodes, edges, params, senders, receivers):
    recv = receivers.astype(jnp.int32)
    send = senders.astype(jnp.int32)

    nodes_t = jnp.zeros((3, NP), jnp.float32).at[:, :N].set(nodes.T)
    edges_t = edges.T

    prep_w = _flat_t(params['pn']) + _flat_t(params['pr'])
    tbl_t, pk_t, dec_t = _tc_prep(nodes_t, prep_w)

    cols = [pk_t[j] for j in range(3)]
    edge_w = _flat_t(params['pe']) + _flat_t(params['em'])
    zeros = jnp.zeros((NP,), jnp.float32)

    agg_all = []
    for m in range(_M):
        sl = slice(m * _EC, (m + 1) * _EC)
        g6 = _sc_gather(cols, recv[sl], send[sl])
        ucols = _tc_edge(edges_t[:, sl], list(g6), edge_w)
        agg_all.extend(_sc_scatter(list(ucols), recv[sl], zeros))

    node_w = _flat_t(params['nm']) + _flat_t(params['gm'])
    res = _tc_node(tbl_t, agg_all, node_w)

    decoded = dec_t[:, :N].T
    return decoded, res[0:9, 2]
